# baseline (speedup 1.0000x reference)
import functools

import jax
import jax.numpy as jnp
from jax import lax
from jax.experimental import pallas as pl
from jax.experimental.pallas import tpu as pltpu
from jax.experimental.pallas import tpu_sc as plsc

_JACCARD_THRESH = 0.5
_NEGPOS_RATIO = 3
_NOBJ = 8
_G = 8
_D = 8732
_DP = 8960
_C = 21
_B = 32
_NF = 10
_CH = 1280
_L = 16


def _sc_match_kernel(tprep_hbm, rows_hbm, dcx_hbm, dcy_hbm, iw_hbm, ih_hbm,
                     lw5_hbm, lh5_hbm, out_hbm,
                     tv, rows_v, planes_v, gat_v, sem):
    wid = lax.axis_index("s") * 2 + lax.axis_index("c")

    toff = pl.multiple_of(wid * (_NF * _NOBJ * _L), 8)
    pltpu.sync_copy(tprep_hbm.at[pl.ds(toff, _NF * _NOBJ * _L)], tv)
    iota = lax.iota(jnp.int32, _L)

    def bvec(j):
        return tv[pl.ds(j * _L, _L)]

    tf = [[bvec(i * _NF + f) for f in range(_NF)] for i in range(_NOBJ)]

    def dgather(v, idx):
        return lax.gather(
            v, idx[:, None],
            lax.GatherDimensionNumbers(offset_dims=(), collapsed_slice_dims=(0,),
                                       start_index_map=(0,)),
            slice_sizes=(1,), mode=lax.GatherScatterMode.PROMISE_IN_BOUNDS)

    def allmax(v):
        for k in (1, 2, 4, 8):
            v = jnp.maximum(v, dgather(v, jnp.bitwise_xor(iota, k)))
        return v

    def allmin_i(v):
        for k in (1, 2, 4, 8):
            v = jnp.minimum(v, dgather(v, jnp.bitwise_xor(iota, k)))
        return v

    neg1 = jnp.float32(-1.0)

    maxv = [jnp.full((_L,), neg1) for _ in range(_NOBJ)]
    idxv = [jnp.zeros((_L,), jnp.int32) for _ in range(_NOBJ)]

    for ch in range(_DP // _CH):
        for r in range(11):
            pltpu.sync_copy(rows_hbm.at[pl.ds(r * _DP + ch * _CH, _CH)],
                            rows_v.at[pl.ds(r * _CH, _CH)])

        def body(j, carry):
            mx = list(carry[:_NOBJ])
            ix = list(carry[_NOBJ:])
            for u in range(4):
                off = j * (4 * _L) + u * _L
                gidx = iota + (ch * _CH + off)
                pf0 = rows_v[pl.ds(0 * _CH + off, _L)]
                pf1 = rows_v[pl.ds(1 * _CH + off, _L)]
                pf2 = rows_v[pl.ds(2 * _CH + off, _L)]
                pf3 = rows_v[pl.ds(3 * _CH + off, _L)]
                area_b = rows_v[pl.ds(4 * _CH + off, _L)]
                dcx = rows_v[pl.ds(5 * _CH + off, _L)]
                dcy = rows_v[pl.ds(6 * _CH + off, _L)]
                i01w = rows_v[pl.ds(7 * _CH + off, _L)]
                i01h = rows_v[pl.ds(8 * _CH + off, _L)]
                lw5 = rows_v[pl.ds(9 * _CH + off, _L)]
                lh5 = rows_v[pl.ds(10 * _CH + off, _L)]

                bto = jnp.full((_L,), neg1)
                bti = jnp.zeros((_L,), jnp.int32)
                for i in range(_NOBJ):
                    x0, y0, x1, y1, aa = tf[i][0], tf[i][1], tf[i][2], tf[i][3], tf[i][4]
                    iw = jnp.maximum(jnp.minimum(x1, pf2) - jnp.maximum(x0, pf0), 0.0)
                    ih = jnp.maximum(jnp.minimum(y1, pf3) - jnp.maximum(y0, pf1), 0.0)
                    inter = iw * ih
                    ov = inter / (aa + area_b - inter)
                    if i == 0:
                        bto = ov
                    else:
                        better = ov > bto
                        bto = jnp.where(better, ov, bto)
                        bti = jnp.where(better, i, bti)
                    upd = ov > mx[i]
                    mx[i] = jnp.where(upd, ov, mx[i])
                    ix[i] = jnp.where(upd, gidx, ix[i])

                s0 = jnp.zeros((_L,), jnp.float32)
                s1 = jnp.zeros((_L,), jnp.float32)
                lw = jnp.zeros((_L,), jnp.float32)
                lh = jnp.zeros((_L,), jnp.float32)
                lab = jnp.zeros((_L,), jnp.float32)
                for i in range(_NOBJ):
                    sel = bti == i
                    s0 = jnp.where(sel, tf[i][5], s0)
                    s1 = jnp.where(sel, tf[i][6], s1)
                    lw = jnp.where(sel, tf[i][7], lw)
                    lh = jnp.where(sel, tf[i][8], lh)
                    lab = jnp.where(sel, tf[i][9], lab)

                ct = jnp.where(bto >= _JACCARD_THRESH, lab + 1.0, 0.0)
                base = ch * _CH + off
                planes_v[pl.ds(0 * _DP + base, _L)] = (s0 - dcx) * i01w
                planes_v[pl.ds(1 * _DP + base, _L)] = (s1 - dcy) * i01h
                planes_v[pl.ds(2 * _DP + base, _L)] = lw - lw5
                planes_v[pl.ds(3 * _DP + base, _L)] = lh - lh5
                planes_v[pl.ds(4 * _DP + base, _L)] = ct
            return tuple(mx) + tuple(ix)

        carry = lax.fori_loop(0, _CH // (4 * _L), body, tuple(maxv) + tuple(idxv))
        maxv = list(carry[:_NOBJ])
        idxv = list(carry[_NOBJ:])

    fidx = jnp.zeros((_L,), jnp.int32)
    for i in range(_NOBJ):
        m = allmax(maxv[i])
        cand = jnp.where(maxv[i] == m, idxv[i], jnp.int32(2 ** 30))
        bp = allmin_i(cand)
        fidx = jnp.where(iota == i, bp, fidx)

    pltpu.async_copy(dcx_hbm.at[fidx], gat_v.at[pl.ds(0 * _L, _L)], sem).wait()
    pltpu.async_copy(dcy_hbm.at[fidx], gat_v.at[pl.ds(1 * _L, _L)], sem).wait()
    pltpu.async_copy(iw_hbm.at[fidx], gat_v.at[pl.ds(2 * _L, _L)], sem).wait()
    pltpu.async_copy(ih_hbm.at[fidx], gat_v.at[pl.ds(3 * _L, _L)], sem).wait()
    pltpu.async_copy(lw5_hbm.at[fidx], gat_v.at[pl.ds(4 * _L, _L)], sem).wait()
    pltpu.async_copy(lh5_hbm.at[fidx], gat_v.at[pl.ds(5 * _L, _L)], sem).wait()

    s0f = jnp.zeros((_L,), jnp.float32)
    s1f = jnp.zeros((_L,), jnp.float32)
    lwf = jnp.zeros((_L,), jnp.float32)
    lhf = jnp.zeros((_L,), jnp.float32)
    labf = jnp.zeros((_L,), jnp.float32)
    for i in range(_NOBJ):
        sel = iota == i
        s0f = jnp.where(sel, tf[i][5], s0f)
        s1f = jnp.where(sel, tf[i][6], s1f)
        lwf = jnp.where(sel, tf[i][7], lwf)
        lhf = jnp.where(sel, tf[i][8], lhf)
        labf = jnp.where(sel, tf[i][9], labf)
    g0f = (s0f - gat_v[pl.ds(0 * _L, _L)]) * gat_v[pl.ds(2 * _L, _L)]
    g1f = (s1f - gat_v[pl.ds(1 * _L, _L)]) * gat_v[pl.ds(3 * _L, _L)]
    g2f = lwf - gat_v[pl.ds(4 * _L, _L)]
    g3f = lhf - gat_v[pl.ds(5 * _L, _L)]
    ctf = labf + 1.0

    for i in range(_NOBJ):
        m = iota == i
        plsc.store_scatter(planes_v.at[pl.ds(0 * _DP, _DP)], [fidx], g0f, mask=m)
        plsc.store_scatter(planes_v.at[pl.ds(1 * _DP, _DP)], [fidx], g1f, mask=m)
        plsc.store_scatter(planes_v.at[pl.ds(2 * _DP, _DP)], [fidx], g2f, mask=m)
        plsc.store_scatter(planes_v.at[pl.ds(3 * _DP, _DP)], [fidx], g3f, mask=m)
        plsc.store_scatter(planes_v.at[pl.ds(4 * _DP, _DP)], [fidx], ctf, mask=m)

    for p in range(5):
        ooff = pl.multiple_of((p * _B + wid) * _DP, 8)
        pltpu.sync_copy(planes_v.at[pl.ds(p * _DP, _DP)], out_hbm.at[pl.ds(ooff, _DP)])


def _sc_match(tprep, rows, enc_rows):
    mesh = plsc.VectorSubcoreMesh(core_axis_name="c", subcore_axis_name="s")
    kfn = functools.partial(
        pl.kernel,
        mesh=mesh,
        compiler_params=pltpu.CompilerParams(needs_layout_passes=False),
        out_type=jax.ShapeDtypeStruct((5 * _B * _DP,), jnp.float32),
        scratch_types=[
            pltpu.VMEM((_NF * _NOBJ * _L,), jnp.float32),
            pltpu.VMEM((11 * _CH,), jnp.float32),
            pltpu.VMEM((5 * _DP,), jnp.float32),
            pltpu.VMEM((6 * _L,), jnp.float32),
            pltpu.SemaphoreType.DMA,
        ],
    )(_sc_match_kernel)
    return kfn(tprep, rows, enc_rows[0], enc_rows[1], enc_rows[2],
               enc_rows[3], enc_rows[4], enc_rows[5])


def _loss_kernel(planes_ref, loc_ref, conf_ref, out_ref):
    b = pl.program_id(0)
    D = loc_ref.shape[3]
    C = conf_ref.shape[1]

    ct = planes_ref[4, 0][:, :D]
    pos = ct > 0.0
    posf = pos.astype(jnp.float32)
    num_pos = jnp.sum(posf, axis=1, keepdims=True)
    conf_t = ct.astype(jnp.int32)

    loss_l = jnp.float32(0.0)
    for r in range(4):
        d = jnp.abs(loc_ref[0, r] - planes_ref[r, 0][:, :D])
        sl1 = jnp.where(d < 1.0, 0.5 * d * d, d - 0.5)
        loss_l = loss_l + jnp.sum(jnp.where(pos, sl1, 0.0))

    ssum = jnp.zeros((_G, D), jnp.float32)
    picked = jnp.zeros((_G, D), jnp.float32)
    for cc in range(C):
        plane = conf_ref[0, cc]
        ssum = ssum + jnp.exp(plane)
        picked = jnp.where(conf_t == cc, plane, picked)
    loss_c = jnp.log(ssum) - picked

    masked = jnp.where(pos, 0.0, loss_c)
    bits = jax.lax.bitcast_convert_type(masked, jnp.int32)
    k = jnp.minimum(num_pos.astype(jnp.int32) * _NEGPOS_RATIO, D)

    def body(_, lohi):
        lo, hi = lohi
        mid = lo + (hi - lo + 1) // 2
        cnt = jnp.sum((bits >= mid).astype(jnp.int32), axis=1, keepdims=True)
        ok = cnt >= k
        return jnp.where(ok, mid, lo), jnp.where(ok, hi, mid - 1)

    lo0 = jnp.zeros((_G, 1), jnp.int32)
    hi0 = jnp.full((_G, 1), 0x7F7FFFFF, jnp.int32)
    lo, _ = jax.lax.fori_loop(0, 31, body, (lo0, hi0))
    vk = jax.lax.bitcast_convert_type(lo, jnp.float32)
    gt = masked > vk
    cnt_gt = jnp.sum(gt.astype(jnp.float32), axis=1, keepdims=True)
    sum_gt = jnp.sum(jnp.where(gt, masked, 0.0), axis=1, keepdims=True)
    kf = k.astype(jnp.float32)
    topk = jnp.where(k > 0, sum_gt + (kf - cnt_gt) * vk, 0.0)
    loss_c_tot = jnp.sum(jnp.where(pos, loss_c, 0.0)) + jnp.sum(topk)

    @pl.when(b == 0)
    def _init():
        out_ref[0, 0] = loss_l
        out_ref[0, 1] = loss_c_tot
        out_ref[0, 2] = jnp.sum(num_pos)

    @pl.when(b != 0)
    def _acc():
        out_ref[0, 0] += loss_l
        out_ref[0, 1] += loss_c_tot
        out_ref[0, 2] += jnp.sum(num_pos)


@jax.jit
def kernel(loc, conf, dbox_list, targets):
    B, D, C = conf.shape
    nb = B // _G

    dcx, dcy = dbox_list[:, 0], dbox_list[:, 1]
    dw, dh = dbox_list[:, 2], dbox_list[:, 3]
    pf0, pf1 = dcx - dw / 2.0, dcy - dh / 2.0
    pf2, pf3 = dcx + dw / 2.0, dcy + dh / 2.0
    rows = jnp.stack([
        pf0, pf1, pf2, pf3,
        (pf2 - pf0) * (pf3 - pf1),
        dcx, dcy,
        1.0 / (0.1 * dw), 1.0 / (0.1 * dh),
        jnp.log(dw) * 5.0, jnp.log(dh) * 5.0,
    ])
    pad = jnp.tile(
        jnp.array([[100.0], [100.0], [-100.0], [-100.0], [1.0],
                   [0.0], [0.0], [1.0], [1.0], [0.0], [0.0]], jnp.float32),
        (1, _DP - D))
    rows_p = jnp.concatenate([rows, pad], axis=1)
    enc_rows = [rows_p[i] for i in range(5, 11)]

    tx0, ty0 = targets[..., 0], targets[..., 1]
    tx1, ty1 = targets[..., 2], targets[..., 3]
    tlab = targets[..., 4]
    tprep = jnp.stack([
        tx0, ty0, tx1, ty1,
        (tx1 - tx0) * (ty1 - ty0),
        (tx0 + tx1) * 0.5, (ty0 + ty1) * 0.5,
        jnp.log(tx1 - tx0) * 5.0, jnp.log(ty1 - ty0) * 5.0,
        tlab,
    ], axis=-1).reshape(B, _NOBJ * _NF)

    tprep_b = jnp.tile(tprep[:, :, None], (1, 1, _L))
    planes = _sc_match(tprep_b.reshape(-1), rows_p.reshape(-1), enc_rows)
    planes_r = planes.reshape(5, nb, _G, _DP)

    loc_r = loc.reshape(nb, _G, D, 4).transpose(0, 3, 1, 2)
    conf_r = conf.reshape(nb, _G, D, C).transpose(0, 3, 1, 2)

    out = pl.pallas_call(
        _loss_kernel,
        grid=(nb,),
        in_specs=[
            pl.BlockSpec((5, 1, _G, _DP), lambda b: (0, b, 0, 0)),
            pl.BlockSpec((1, 4, _G, D), lambda b: (b, 0, 0, 0)),
            pl.BlockSpec((1, C, _G, D), lambda b: (b, 0, 0, 0)),
        ],
        out_specs=pl.BlockSpec((1, 3), lambda b: (0, 0), memory_space=pltpu.SMEM),
        out_shape=jax.ShapeDtypeStruct((1, 3), jnp.float32),
    )(planes_r, loc_r, conf_r)

    N = out[0, 2]
    return (out[0, 0] / N, out[0, 1] / N)


# speedup vs baseline: 1.3011x; 1.3011x over previous
import functools

import jax
import jax.numpy as jnp
from jax import lax
from jax.experimental import pallas as pl
from jax.experimental.pallas import tpu as pltpu
from jax.experimental.pallas import tpu_sc as plsc

_JACCARD_THRESH = 0.5
_NEGPOS_RATIO = 3
_NOBJ = 8
_G = 8
_D = 8732
_DP = 8960
_C = 21
_B = 32
_NF = 10
_CH = 1280
_L = 16


def _sc_match_kernel(tprep_hbm, rows_hbm, dcx_hbm, dcy_hbm, iw_hbm, ih_hbm,
                     lw5_hbm, lh5_hbm, out_hbm,
                     tv, rows_v, planes_v, gat_v, sem):
    wid = lax.axis_index("s") * 2 + lax.axis_index("c")

    toff = pl.multiple_of(wid * (_NF * _NOBJ * _L), 8)
    pltpu.sync_copy(tprep_hbm.at[pl.ds(toff, _NF * _NOBJ * _L)], tv)
    iota = lax.iota(jnp.int32, _L)

    def bvec(j):
        return tv[pl.ds(j * _L, _L)]

    tf = [[bvec(i * _NF + f) for f in range(_NF)] for i in range(_NOBJ)]

    def dgather(v, idx):
        return lax.gather(
            v, idx[:, None],
            lax.GatherDimensionNumbers(offset_dims=(), collapsed_slice_dims=(0,),
                                       start_index_map=(0,)),
            slice_sizes=(1,), mode=lax.GatherScatterMode.PROMISE_IN_BOUNDS)

    def allmax(v):
        for k in (1, 2, 4, 8):
            v = jnp.maximum(v, dgather(v, jnp.bitwise_xor(iota, k)))
        return v

    def allmin_i(v):
        for k in (1, 2, 4, 8):
            v = jnp.minimum(v, dgather(v, jnp.bitwise_xor(iota, k)))
        return v

    neg1 = jnp.float32(-1.0)

    maxv = [jnp.full((_L,), neg1) for _ in range(_NOBJ)]
    idxv = [jnp.zeros((_L,), jnp.int32) for _ in range(_NOBJ)]

    for ch in range(_DP // _CH):
        pltpu.sync_copy(rows_hbm.at[pl.ds(ch * 11 * _CH, 11 * _CH)], rows_v)

        def body(j, carry):
            mx = list(carry[:_NOBJ])
            ix = list(carry[_NOBJ:])
            for u in range(4):
                off = j * (4 * _L) + u * _L
                gidx = iota + (ch * _CH + off)
                pf0 = rows_v[pl.ds(0 * _CH + off, _L)]
                pf1 = rows_v[pl.ds(1 * _CH + off, _L)]
                pf2 = rows_v[pl.ds(2 * _CH + off, _L)]
                pf3 = rows_v[pl.ds(3 * _CH + off, _L)]
                area_b = rows_v[pl.ds(4 * _CH + off, _L)]
                dcx = rows_v[pl.ds(5 * _CH + off, _L)]
                dcy = rows_v[pl.ds(6 * _CH + off, _L)]
                i01w = rows_v[pl.ds(7 * _CH + off, _L)]
                i01h = rows_v[pl.ds(8 * _CH + off, _L)]
                lw5 = rows_v[pl.ds(9 * _CH + off, _L)]
                lh5 = rows_v[pl.ds(10 * _CH + off, _L)]

                bto = jnp.full((_L,), neg1)
                bti = jnp.zeros((_L,), jnp.int32)
                for i in range(_NOBJ):
                    x0, y0, x1, y1, aa = tf[i][0], tf[i][1], tf[i][2], tf[i][3], tf[i][4]
                    iw = jnp.maximum(jnp.minimum(x1, pf2) - jnp.maximum(x0, pf0), 0.0)
                    ih = jnp.maximum(jnp.minimum(y1, pf3) - jnp.maximum(y0, pf1), 0.0)
                    inter = iw * ih
                    ov = inter / (aa + area_b - inter)
                    if i == 0:
                        bto = ov
                    else:
                        better = ov > bto
                        bto = jnp.where(better, ov, bto)
                        bti = jnp.where(better, i, bti)
                    upd = ov > mx[i]
                    mx[i] = jnp.where(upd, ov, mx[i])
                    ix[i] = jnp.where(upd, gidx, ix[i])

                s0 = jnp.zeros((_L,), jnp.float32)
                s1 = jnp.zeros((_L,), jnp.float32)
                lw = jnp.zeros((_L,), jnp.float32)
                lh = jnp.zeros((_L,), jnp.float32)
                lab = jnp.zeros((_L,), jnp.float32)
                for i in range(_NOBJ):
                    sel = bti == i
                    s0 = jnp.where(sel, tf[i][5], s0)
                    s1 = jnp.where(sel, tf[i][6], s1)
                    lw = jnp.where(sel, tf[i][7], lw)
                    lh = jnp.where(sel, tf[i][8], lh)
                    lab = jnp.where(sel, tf[i][9], lab)

                ct = jnp.where(bto >= _JACCARD_THRESH, lab + 1.0, 0.0)
                base = ch * _CH + off
                planes_v[pl.ds(0 * _DP + base, _L)] = (s0 - dcx) * i01w
                planes_v[pl.ds(1 * _DP + base, _L)] = (s1 - dcy) * i01h
                planes_v[pl.ds(2 * _DP + base, _L)] = lw - lw5
                planes_v[pl.ds(3 * _DP + base, _L)] = lh - lh5
                planes_v[pl.ds(4 * _DP + base, _L)] = ct
            return tuple(mx) + tuple(ix)

        carry = lax.fori_loop(0, _CH // (4 * _L), body, tuple(maxv) + tuple(idxv))
        maxv = list(carry[:_NOBJ])
        idxv = list(carry[_NOBJ:])

    fidx = jnp.zeros((_L,), jnp.int32)
    for i in range(_NOBJ):
        m = allmax(maxv[i])
        cand = jnp.where(maxv[i] == m, idxv[i], jnp.int32(2 ** 30))
        bp = allmin_i(cand)
        fidx = jnp.where(iota == i, bp, fidx)

    pltpu.async_copy(dcx_hbm.at[fidx], gat_v.at[pl.ds(0 * _L, _L)], sem).wait()
    pltpu.async_copy(dcy_hbm.at[fidx], gat_v.at[pl.ds(1 * _L, _L)], sem).wait()
    pltpu.async_copy(iw_hbm.at[fidx], gat_v.at[pl.ds(2 * _L, _L)], sem).wait()
    pltpu.async_copy(ih_hbm.at[fidx], gat_v.at[pl.ds(3 * _L, _L)], sem).wait()
    pltpu.async_copy(lw5_hbm.at[fidx], gat_v.at[pl.ds(4 * _L, _L)], sem).wait()
    pltpu.async_copy(lh5_hbm.at[fidx], gat_v.at[pl.ds(5 * _L, _L)], sem).wait()

    s0f = jnp.zeros((_L,), jnp.float32)
    s1f = jnp.zeros((_L,), jnp.float32)
    lwf = jnp.zeros((_L,), jnp.float32)
    lhf = jnp.zeros((_L,), jnp.float32)
    labf = jnp.zeros((_L,), jnp.float32)
    for i in range(_NOBJ):
        sel = iota == i
        s0f = jnp.where(sel, tf[i][5], s0f)
        s1f = jnp.where(sel, tf[i][6], s1f)
        lwf = jnp.where(sel, tf[i][7], lwf)
        lhf = jnp.where(sel, tf[i][8], lhf)
        labf = jnp.where(sel, tf[i][9], labf)
    g0f = (s0f - gat_v[pl.ds(0 * _L, _L)]) * gat_v[pl.ds(2 * _L, _L)]
    g1f = (s1f - gat_v[pl.ds(1 * _L, _L)]) * gat_v[pl.ds(3 * _L, _L)]
    g2f = lwf - gat_v[pl.ds(4 * _L, _L)]
    g3f = lhf - gat_v[pl.ds(5 * _L, _L)]
    ctf = labf + 1.0

    for i in range(_NOBJ):
        m = iota == i
        plsc.store_scatter(planes_v.at[pl.ds(0 * _DP, _DP)], [fidx], g0f, mask=m)
        plsc.store_scatter(planes_v.at[pl.ds(1 * _DP, _DP)], [fidx], g1f, mask=m)
        plsc.store_scatter(planes_v.at[pl.ds(2 * _DP, _DP)], [fidx], g2f, mask=m)
        plsc.store_scatter(planes_v.at[pl.ds(3 * _DP, _DP)], [fidx], g3f, mask=m)
        plsc.store_scatter(planes_v.at[pl.ds(4 * _DP, _DP)], [fidx], ctf, mask=m)

    for p in range(5):
        ooff = pl.multiple_of((p * _B + wid) * _DP, 8)
        pltpu.sync_copy(planes_v.at[pl.ds(p * _DP, _DP)], out_hbm.at[pl.ds(ooff, _DP)])


def _sc_match(tprep, rows, enc_rows):
    mesh = plsc.VectorSubcoreMesh(core_axis_name="c", subcore_axis_name="s")
    kfn = functools.partial(
        pl.kernel,
        mesh=mesh,
        compiler_params=pltpu.CompilerParams(needs_layout_passes=False),
        out_type=jax.ShapeDtypeStruct((5 * _B * _DP,), jnp.float32),
        scratch_types=[
            pltpu.VMEM((_NF * _NOBJ * _L,), jnp.float32),
            pltpu.VMEM((11 * _CH,), jnp.float32),
            pltpu.VMEM((5 * _DP,), jnp.float32),
            pltpu.VMEM((6 * _L,), jnp.float32),
            pltpu.SemaphoreType.DMA,
        ],
    )(_sc_match_kernel)
    return kfn(tprep, rows, enc_rows[0], enc_rows[1], enc_rows[2],
               enc_rows[3], enc_rows[4], enc_rows[5])


def _loss_kernel(planes_ref, loc_ref, conf_ref, out_ref):
    b = pl.program_id(0)
    D = loc_ref.shape[3]
    C = conf_ref.shape[1]

    ct = planes_ref[4, 0][:, :D]
    pos = ct > 0.0
    posf = pos.astype(jnp.float32)
    num_pos = jnp.sum(posf, axis=1, keepdims=True)
    conf_t = ct.astype(jnp.int32)

    loss_l = jnp.float32(0.0)
    for r in range(4):
        d = jnp.abs(loc_ref[0, r] - planes_ref[r, 0][:, :D])
        sl1 = jnp.where(d < 1.0, 0.5 * d * d, d - 0.5)
        loss_l = loss_l + jnp.sum(jnp.where(pos, sl1, 0.0))

    ssum = jnp.zeros((_G, D), jnp.float32)
    picked = jnp.zeros((_G, D), jnp.float32)
    for cc in range(C):
        plane = conf_ref[0, cc]
        ssum = ssum + jnp.exp(plane)
        picked = jnp.where(conf_t == cc, plane, picked)
    loss_c = jnp.log(ssum) - picked

    masked = jnp.where(pos, 0.0, loss_c)
    bits = jax.lax.bitcast_convert_type(masked, jnp.int32)
    k = jnp.minimum(num_pos.astype(jnp.int32) * _NEGPOS_RATIO, D)

    def body(_, lohi):
        lo, hi = lohi
        mid = lo + (hi - lo + 1) // 2
        cnt = jnp.sum((bits >= mid).astype(jnp.int32), axis=1, keepdims=True)
        ok = cnt >= k
        return jnp.where(ok, mid, lo), jnp.where(ok, hi, mid - 1)

    lo0 = jnp.zeros((_G, 1), jnp.int32)
    hi0 = jnp.full((_G, 1), 0x7F7FFFFF, jnp.int32)
    lo, _ = jax.lax.fori_loop(0, 31, body, (lo0, hi0))
    vk = jax.lax.bitcast_convert_type(lo, jnp.float32)
    gt = masked > vk
    cnt_gt = jnp.sum(gt.astype(jnp.float32), axis=1, keepdims=True)
    sum_gt = jnp.sum(jnp.where(gt, masked, 0.0), axis=1, keepdims=True)
    kf = k.astype(jnp.float32)
    topk = jnp.where(k > 0, sum_gt + (kf - cnt_gt) * vk, 0.0)
    loss_c_tot = jnp.sum(jnp.where(pos, loss_c, 0.0)) + jnp.sum(topk)

    @pl.when(b == 0)
    def _init():
        out_ref[0, 0] = loss_l
        out_ref[0, 1] = loss_c_tot
        out_ref[0, 2] = jnp.sum(num_pos)

    @pl.when(b != 0)
    def _acc():
        out_ref[0, 0] += loss_l
        out_ref[0, 1] += loss_c_tot
        out_ref[0, 2] += jnp.sum(num_pos)


@jax.jit
def kernel(loc, conf, dbox_list, targets):
    B, D, C = conf.shape
    nb = B // _G

    dcx, dcy = dbox_list[:, 0], dbox_list[:, 1]
    dw, dh = dbox_list[:, 2], dbox_list[:, 3]
    pf0, pf1 = dcx - dw / 2.0, dcy - dh / 2.0
    pf2, pf3 = dcx + dw / 2.0, dcy + dh / 2.0
    rows = jnp.stack([
        pf0, pf1, pf2, pf3,
        (pf2 - pf0) * (pf3 - pf1),
        dcx, dcy,
        1.0 / (0.1 * dw), 1.0 / (0.1 * dh),
        jnp.log(dw) * 5.0, jnp.log(dh) * 5.0,
    ])
    pad = jnp.tile(
        jnp.array([[100.0], [100.0], [-100.0], [-100.0], [1.0],
                   [0.0], [0.0], [1.0], [1.0], [0.0], [0.0]], jnp.float32),
        (1, _DP - D))
    rows_p = jnp.concatenate([rows, pad], axis=1)
    enc_rows = [rows_p[i] for i in range(5, 11)]

    tx0, ty0 = targets[..., 0], targets[..., 1]
    tx1, ty1 = targets[..., 2], targets[..., 3]
    tlab = targets[..., 4]
    tprep = jnp.stack([
        tx0, ty0, tx1, ty1,
        (tx1 - tx0) * (ty1 - ty0),
        (tx0 + tx1) * 0.5, (ty0 + ty1) * 0.5,
        jnp.log(tx1 - tx0) * 5.0, jnp.log(ty1 - ty0) * 5.0,
        tlab,
    ], axis=-1).reshape(B, _NOBJ * _NF)

    tprep_b = jnp.tile(tprep[:, :, None], (1, 1, _L))
    rows_c = rows_p.reshape(11, _DP // _CH, _CH).swapaxes(0, 1)
    planes = _sc_match(tprep_b.reshape(-1), rows_c.reshape(-1), enc_rows)
    planes_r = planes.reshape(5, nb, _G, _DP)

    loc_r = loc.reshape(nb, _G, D, 4).transpose(0, 3, 1, 2)
    conf_r = conf.reshape(nb, _G, D, C).transpose(0, 3, 1, 2)

    out = pl.pallas_call(
        _loss_kernel,
        grid=(nb,),
        in_specs=[
            pl.BlockSpec((5, 1, _G, _DP), lambda b: (0, b, 0, 0)),
            pl.BlockSpec((1, 4, _G, D), lambda b: (b, 0, 0, 0)),
            pl.BlockSpec((1, C, _G, D), lambda b: (b, 0, 0, 0)),
        ],
        out_specs=pl.BlockSpec((1, 3), lambda b: (0, 0), memory_space=pltpu.SMEM),
        out_shape=jax.ShapeDtypeStruct((1, 3), jnp.float32),
    )(planes_r, loc_r, conf_r)

    N = out[0, 2]
    return (out[0, 0] / N, out[0, 1] / N)
